# Initial kernel scaffold; baseline (speedup 1.0000x reference)
import jax
import jax.numpy as jnp
from jax.experimental import pallas as pl


def kernel(pred, target):
    raise NotImplementedError("write your pallas kernel here")


# speedup vs baseline: 2.0769x; 2.0769x over previous
import jax
import jax.numpy as jnp
from jax.experimental import pallas as pl
from jax.experimental.pallas import tpu as pltpu
import functools

N = 8192
M = 8192
BJ = 512
NJ = M // BJ


def _chamfer_body(pred_ref, tgt_ref, out_ref, rowmin_ref, colsum_ref):
    j = pl.program_id(0)

    px = pred_ref[:, 0:1]
    py = pred_ref[:, 1:2]
    pz = pred_ref[:, 2:3]
    tx = tgt_ref[0:1, :]
    ty = tgt_ref[1:2, :]
    tz = tgt_ref[2:3, :]

    dx = px - tx
    dy = py - ty
    dz = pz - tz
    d2 = dx * dx + dy * dy + dz * dz

    block_rowmin = jnp.min(d2, axis=1, keepdims=True)
    colmin = jnp.min(d2, axis=0, keepdims=True)

    @pl.when(j == 0)
    def _init():
        rowmin_ref[:, :] = block_rowmin
        colsum_ref[0, 0] = 0.0

    @pl.when(j > 0)
    def _acc():
        rowmin_ref[:, :] = jnp.minimum(rowmin_ref[:, :], block_rowmin)

    colsum_ref[0, 0] += jnp.sum(jnp.sqrt(colmin))

    @pl.when(j == NJ - 1)
    def _final():
        rowsum = jnp.sum(jnp.sqrt(rowmin_ref[:, :]))
        out_ref[0, 0] = rowsum * (1.0 / N) + colsum_ref[0, 0] * (1.0 / M)


@jax.jit
def kernel(pred, target):
    tgt_t = target.T
    out = pl.pallas_call(
        _chamfer_body,
        grid=(NJ,),
        in_specs=[
            pl.BlockSpec((N, 3), lambda j: (0, 0)),
            pl.BlockSpec((3, BJ), lambda j: (0, j)),
        ],
        out_specs=pl.BlockSpec((1, 1), lambda j: (0, 0), memory_space=pltpu.SMEM),
        out_shape=jax.ShapeDtypeStruct((1, 1), jnp.float32),
        scratch_shapes=[
            pltpu.VMEM((N, 1), jnp.float32),
            pltpu.SMEM((1, 1), jnp.float32),
        ],
    )(pred, tgt_t)
    return out[0, 0]
